# baseline (speedup 1.0000x reference)
import functools

import jax
import jax.numpy as jnp
from jax import lax
from jax.experimental import pallas as pl
from jax.experimental.pallas import tpu as pltpu
from jax.experimental.pallas import tpu_sc as plsc

N = 8192
C = 768
H = 12
CH = C // H
Z = 512
M = 256
D = 2
SCALE = CH ** -0.5

NCORES = 2
NSUB = 16
NWORKERS = NCORES * NSUB
RPC = 8
NCHUNKS = Z // RPC


def _sc_body(mi_hbm, cs_hbm, mask_hbm, s_hbm, msum_hbm,
             mi_v, cs_v, mask_v, s_v, msum_v):
    cid = lax.axis_index("c")
    sid = lax.axis_index("s")
    wid = sid * NCORES + cid
    lanes = lax.iota(jnp.int32, 16)
    lm = lanes < RPC
    zeros16 = jnp.zeros((16,), jnp.float32)
    pltpu.sync_copy(mask_hbm, mask_v)

    nreps = NCHUNKS // NWORKERS
    for rep in range(nreps):
        c = wid + NWORKERS * rep
        base = c * RPC
        pltpu.sync_copy(mi_hbm.at[pl.ds(base * M, RPC * M)], mi_v)
        pltpu.sync_copy(cs_hbm.at[pl.ds(base * M, RPC * M)], cs_v)

        if rep == 0:
            def zbody(j, carry):
                s_v[pl.ds(j * 16, 16)] = zeros16
                return carry
            lax.fori_loop(0, RPC * N // 16, zbody, 0, unroll=8)

        def mbody(m, acc):
            flat = lanes * M + m
            idx = plsc.load_gather(mi_v, [flat], mask=lm)
            idx = jnp.where(lm, idx, 0)
            sc = plsc.load_gather(cs_v, [flat], mask=lm)
            mval = plsc.load_gather(mask_v, [idx], mask=lm)
            val = sc * mval
            plsc.addupdate_scatter(s_v, [lanes * N + idx], val, mask=lm)
            return acc + jnp.where(lm, mval, 0.0)
        msum = lax.fori_loop(0, M, mbody, zeros16, unroll=4)
        msum_v[...] = msum
        pltpu.sync_copy(s_v, s_hbm.at[pl.ds(base * N, RPC * N)])
        pltpu.sync_copy(msum_v.at[pl.ds(0, RPC)], msum_hbm.at[pl.ds(base, RPC)])

        if rep != nreps - 1:
            def rbody(m, carry):
                flat = lanes * M + m
                idx = plsc.load_gather(mi_v, [flat], mask=lm)
                idx = jnp.where(lm, idx, 0)
                plsc.store_scatter(s_v, [lanes * N + idx], zeros16, mask=lm)
                return carry
            lax.fori_loop(0, M, rbody, 0, unroll=4)

@functools.cache
def _get_sc_build():
    return functools.partial(
        pl.kernel,
        mesh=plsc.VectorSubcoreMesh(core_axis_name="c", subcore_axis_name="s"),
        out_type=(
            jax.ShapeDtypeStruct((Z * N,), jnp.float32),
            jax.ShapeDtypeStruct((Z,), jnp.float32),
        ),
        scratch_types=[
            pltpu.VMEM((RPC * M,), jnp.int32),
            pltpu.VMEM((RPC * M,), jnp.float32),
            pltpu.VMEM((N,), jnp.float32),
            pltpu.VMEM((RPC * N,), jnp.float32),
            pltpu.VMEM((16,), jnp.float32),
        ],
        compiler_params=pltpu.CompilerParams(needs_layout_passes=False),
    )(_sc_body)


NBLK1 = 512
NSTEPS_A = N // NBLK1


def _tc12_body(k_ref, msum_ref, s_ref, feat_ref, q_ref, pos_ref, wk_ref,
               wv_ref, bqkv_ref, wpos_ref, wproj_ref, bproj_ref, out_ref,
               acc_feat, acc_p, acc_cs, acc_max, kr_s, vr_s, bias_s, o_v,
               wproj_b):
    i = pl.program_id(0)

    @pl.when(i == 0)
    def _():
        acc_feat[...] = jnp.zeros_like(acc_feat)
        acc_p[...] = jnp.zeros_like(acc_p)
        acc_cs[...] = jnp.zeros_like(acc_cs)
        acc_max[...] = jnp.full_like(acc_max, -jnp.inf)
        wproj_b[...] = wproj_ref[...].astype(jnp.bfloat16)

    @pl.when(i < NSTEPS_A)
    def _():
        s = s_ref[...]
        p = pos_ref[...]
        acc_feat[...] += lax.dot(s.astype(jnp.bfloat16),
                                 feat_ref[...].astype(jnp.bfloat16),
                                 preferred_element_type=jnp.float32)
        acc_p[...] += lax.dot(s, p, preferred_element_type=jnp.float32)
        acc_cs[...] += jnp.sum(s, axis=1, keepdims=True)
        acc_max[...] = jnp.maximum(acc_max[...],
                                   jnp.max(p, axis=0, keepdims=True))

    @pl.when(i == NSTEPS_A - 1)
    def _():
        fr = acc_feat[...]
        cs = acc_cs[...]
        kr_s[...] = (lax.dot(fr, wk_ref[...],
                             preferred_element_type=jnp.float32)
                     + cs * bqkv_ref[1:2, :]).astype(jnp.bfloat16)
        vr_s[...] = (lax.dot(fr, wv_ref[...],
                             preferred_element_type=jnp.float32)
                     + cs * bqkv_ref[2:3, :]).astype(jnp.bfloat16)
        pos_r = acc_p[...] / acc_max[...]
        bias = lax.dot(pos_r, wpos_ref[...],
                       preferred_element_type=jnp.float32)
        zrow = lax.broadcasted_iota(jnp.int32, (Z, 1), 0)
        mask2 = (msum_ref[...] > 0.0) & (zrow < k_ref[0, 0])
        bias_s[...] = jnp.transpose(bias + jnp.where(mask2, 0.0, -100.0))

    @pl.when(i >= NSTEPS_A)
    def _():
        for h in range(H):
            qh = q_ref[:, h * CH:(h + 1) * CH].astype(jnp.bfloat16)
            kh = kr_s[:, h * CH:(h + 1) * CH]
            logits = lax.dot_general(qh, kh, (((1,), (1,)), ((), ())),
                                     preferred_element_type=jnp.float32)
            logits = logits + bias_s[h, :][None, :]
            mx = jnp.max(logits, axis=1, keepdims=True)
            e = jnp.exp(logits - mx)
            r = 1.0 / jnp.sum(e, axis=1, keepdims=True)
            o_v[:, h * CH:(h + 1) * CH] = (lax.dot(
                e.astype(jnp.bfloat16), vr_s[:, h * CH:(h + 1) * CH],
                preferred_element_type=jnp.float32) * r
            ).astype(jnp.bfloat16)
        out_ref[...] = lax.dot(o_v[...], wproj_b[...],
                               preferred_element_type=jnp.float32) \
            + bproj_ref[...]


def _ia(i):
    return jnp.minimum(i, NSTEPS_A - 1)


def _ib(i):
    return jnp.maximum(i - NSTEPS_A, 0)


_tc12 = pl.pallas_call(
    _tc12_body,
    grid=(2 * NSTEPS_A,),
    in_specs=[
        pl.BlockSpec(memory_space=pltpu.SMEM),
        pl.BlockSpec((Z, 1), lambda i: (0, 0)),
        pl.BlockSpec((Z, NBLK1), lambda i: (0, _ia(i))),
        pl.BlockSpec((NBLK1, C), lambda i: (_ia(i), 0)),
        pl.BlockSpec((NBLK1, C), lambda i: (_ib(i), 0)),
        pl.BlockSpec((NBLK1, D), lambda i: (_ia(i), 0)),
        pl.BlockSpec((C, C), lambda i: (0, 1)),
        pl.BlockSpec((C, C), lambda i: (0, 2)),
        pl.BlockSpec((3, C), lambda i: (0, 0)),
        pl.BlockSpec((D, H), lambda i: (0, 0)),
        pl.BlockSpec((C, C), lambda i: (0, 0)),
        pl.BlockSpec((1, C), lambda i: (0, 0)),
    ],
    out_specs=pl.BlockSpec((NBLK1, C), lambda i: (_ib(i), 0)),
    out_shape=jax.ShapeDtypeStruct((N, C), jnp.float32),
    scratch_shapes=[
        pltpu.VMEM((Z, C), jnp.float32),
        pltpu.VMEM((Z, D), jnp.float32),
        pltpu.VMEM((Z, 1), jnp.float32),
        pltpu.VMEM((1, D), jnp.float32),
        pltpu.VMEM((Z, C), jnp.bfloat16),
        pltpu.VMEM((Z, C), jnp.bfloat16),
        pltpu.VMEM((H, Z), jnp.float32),
        pltpu.VMEM((NBLK1, C), jnp.bfloat16),
        pltpu.VMEM((C, C), jnp.bfloat16),
    ],
    compiler_params=pltpu.CompilerParams(
        dimension_semantics=("arbitrary",),
        vmem_limit_bytes=100 * 1024 * 1024),
)


NBLKQ = 1024


def _tcq_body(feat_ref, wq_ref, bqkv_ref, q_ref):
    q_ref[...] = (lax.dot(feat_ref[...], wq_ref[...],
                          preferred_element_type=jnp.float32)
                  + bqkv_ref[0:1, :]) * SCALE


_tcq = pl.pallas_call(
    _tcq_body,
    grid=(N // NBLKQ,),
    in_specs=[
        pl.BlockSpec((NBLKQ, C), lambda i: (i, 0)),
        pl.BlockSpec((C, C), lambda i: (0, 0)),
        pl.BlockSpec((3, C), lambda i: (0, 0)),
    ],
    out_specs=pl.BlockSpec((NBLKQ, C), lambda i: (i, 0)),
    out_shape=jax.ShapeDtypeStruct((N, C), jnp.float32),
    compiler_params=pltpu.CompilerParams(
        dimension_semantics=("arbitrary",)),
)


def kernel(pos, feat, cluster_feat, cluster_score, mean_assignment, mask,
           member_idx, batch_idx, k, valid_row_idx, attend_means,
           W_qkv, b_qkv, W_pos, b_pos, W_proj, b_proj):
    feat2 = feat.reshape(N, C)
    pos2 = pos.reshape(N, D)
    maskf = mask.reshape(N).astype(jnp.float32)
    mi = member_idx.astype(jnp.int32)
    cs = cluster_score.astype(jnp.float32)

    s_flat, msum = _get_sc_build()(mi.reshape(-1), cs.reshape(-1), maskf)

    kk = jnp.asarray(k, jnp.int32).reshape(1, 1)
    bqkv3 = b_qkv.reshape(3, C)
    q2 = _tcq(feat2, W_qkv, bqkv3)
    out = _tc12(kk, msum.reshape(Z, 1), s_flat.reshape(Z, N), feat2, q2,
                pos2, W_qkv, W_qkv, bqkv3, W_pos, W_proj,
                b_proj.reshape(1, C))
    return out.reshape(1, N, C)


# speedup vs baseline: 1.1983x; 1.1983x over previous
import functools

import jax
import jax.numpy as jnp
from jax import lax
from jax.experimental import pallas as pl
from jax.experimental.pallas import tpu as pltpu
from jax.experimental.pallas import tpu_sc as plsc

N = 8192
C = 768
H = 12
CH = C // H
Z = 512
M = 256
D = 2
SCALE = CH ** -0.5

NCORES = 2
NSUB = 16
NWORKERS = NCORES * NSUB
RPC = 8
NCHUNKS = Z // RPC


def _sc_body(mi_hbm, cs_hbm, mask_hbm, s_hbm, msum_hbm,
             mi_v, cs_v, mask_v, s_v, msum_v):
    cid = lax.axis_index("c")
    sid = lax.axis_index("s")
    wid = sid * NCORES + cid
    lanes = lax.iota(jnp.int32, 16)
    lm = lanes < RPC
    zeros16 = jnp.zeros((16,), jnp.float32)
    pltpu.sync_copy(mask_hbm, mask_v)

    nreps = NCHUNKS // NWORKERS
    for rep in range(nreps):
        c = wid + NWORKERS * rep
        base = c * RPC
        pltpu.sync_copy(mi_hbm.at[pl.ds(base * M, RPC * M)], mi_v)
        pltpu.sync_copy(cs_hbm.at[pl.ds(base * M, RPC * M)], cs_v)

        if rep == 0:
            def zbody(j, carry):
                s_v[pl.ds(j * 16, 16)] = zeros16
                return carry
            lax.fori_loop(0, RPC * N // 16, zbody, 0, unroll=8)

        def mbody(m, acc):
            flat = lanes * M + m
            idx = plsc.load_gather(mi_v, [flat], mask=lm)
            idx = jnp.where(lm, idx, 0)
            sc = plsc.load_gather(cs_v, [flat], mask=lm)
            mval = plsc.load_gather(mask_v, [idx], mask=lm)
            val = sc * mval
            plsc.addupdate_scatter(s_v, [lanes * N + idx], val, mask=lm)
            return acc + jnp.where(lm, mval, 0.0)
        msum = lax.fori_loop(0, M, mbody, zeros16, unroll=4)
        msum_v[...] = msum
        pltpu.sync_copy(s_v, s_hbm.at[pl.ds(base * N, RPC * N)])
        pltpu.sync_copy(msum_v.at[pl.ds(0, RPC)], msum_hbm.at[pl.ds(base, RPC)])

        if rep != nreps - 1:
            def rbody(m, carry):
                flat = lanes * M + m
                idx = plsc.load_gather(mi_v, [flat], mask=lm)
                idx = jnp.where(lm, idx, 0)
                plsc.store_scatter(s_v, [lanes * N + idx], zeros16, mask=lm)
                return carry
            lax.fori_loop(0, M, rbody, 0, unroll=4)

@functools.cache
def _get_sc_build():
    return functools.partial(
        pl.kernel,
        mesh=plsc.VectorSubcoreMesh(core_axis_name="c", subcore_axis_name="s"),
        out_type=(
            jax.ShapeDtypeStruct((Z * N,), jnp.float32),
            jax.ShapeDtypeStruct((Z,), jnp.float32),
        ),
        scratch_types=[
            pltpu.VMEM((RPC * M,), jnp.int32),
            pltpu.VMEM((RPC * M,), jnp.float32),
            pltpu.VMEM((N,), jnp.float32),
            pltpu.VMEM((RPC * N,), jnp.float32),
            pltpu.VMEM((16,), jnp.float32),
        ],
        compiler_params=pltpu.CompilerParams(needs_layout_passes=False),
    )(_sc_body)


NBLK1 = 1024
NSTEPS_A = N // NBLK1


def _tc12_body(k_ref, msum_ref, s_ref, feat_ref, q_ref, pos_ref, wk_ref,
               wv_ref, bqkv_ref, wpos_ref, wproj_ref, bproj_ref, out_ref,
               acc_feat, acc_p, acc_cs, acc_max, kr_s, vr_s, bias_s, o_v,
               wproj_b):
    i = pl.program_id(0)

    @pl.when(i == 0)
    def _():
        acc_feat[...] = jnp.zeros_like(acc_feat)
        acc_p[...] = jnp.zeros_like(acc_p)
        acc_cs[...] = jnp.zeros_like(acc_cs)
        acc_max[...] = jnp.full_like(acc_max, -jnp.inf)
        wproj_b[...] = wproj_ref[...].astype(jnp.bfloat16)

    @pl.when(i < NSTEPS_A)
    def _():
        s = s_ref[...]
        p = pos_ref[...]
        acc_feat[...] += lax.dot(s.astype(jnp.bfloat16),
                                 feat_ref[...].astype(jnp.bfloat16),
                                 preferred_element_type=jnp.float32)
        acc_p[...] += lax.dot(s, p, preferred_element_type=jnp.float32)
        acc_cs[...] += jnp.sum(s, axis=1, keepdims=True)
        acc_max[...] = jnp.maximum(acc_max[...],
                                   jnp.max(p, axis=0, keepdims=True))

    @pl.when(i == NSTEPS_A - 1)
    def _():
        fr = acc_feat[...]
        cs = acc_cs[...]
        kr_s[...] = (lax.dot(fr, wk_ref[...],
                             preferred_element_type=jnp.float32)
                     + cs * bqkv_ref[1:2, :]).astype(jnp.bfloat16)
        vr_s[...] = (lax.dot(fr, wv_ref[...],
                             preferred_element_type=jnp.float32)
                     + cs * bqkv_ref[2:3, :]).astype(jnp.bfloat16)
        pos_r = acc_p[...] / acc_max[...]
        bias = lax.dot(pos_r, wpos_ref[...],
                       preferred_element_type=jnp.float32)
        zrow = lax.broadcasted_iota(jnp.int32, (Z, 1), 0)
        mask2 = (msum_ref[...] > 0.0) & (zrow < k_ref[0, 0])
        bias_s[...] = jnp.transpose(bias + jnp.where(mask2, 0.0, -100.0))

    @pl.when(i >= NSTEPS_A)
    def _():
        for h in range(H):
            qh = q_ref[:, h * CH:(h + 1) * CH]
            kh = kr_s[:, h * CH:(h + 1) * CH]
            logits = lax.dot_general(qh, kh, (((1,), (1,)), ((), ())),
                                     preferred_element_type=jnp.float32)
            logits = logits + bias_s[h, :][None, :]
            mx = jnp.max(logits, axis=1, keepdims=True)
            e = jnp.exp(logits - mx)
            r = 1.0 / jnp.sum(e, axis=1, keepdims=True)
            o_v[:, h * CH:(h + 1) * CH] = (lax.dot(
                e.astype(jnp.bfloat16), vr_s[:, h * CH:(h + 1) * CH],
                preferred_element_type=jnp.float32) * r
            ).astype(jnp.bfloat16)
        out_ref[...] = lax.dot(o_v[...], wproj_b[...],
                               preferred_element_type=jnp.float32) \
            + bproj_ref[...]


def _ia(i):
    return jnp.minimum(i, NSTEPS_A - 1)


def _ib(i):
    return jnp.maximum(i - NSTEPS_A, 0)


_tc12 = pl.pallas_call(
    _tc12_body,
    grid=(2 * NSTEPS_A,),
    in_specs=[
        pl.BlockSpec(memory_space=pltpu.SMEM),
        pl.BlockSpec((Z, 1), lambda i: (0, 0)),
        pl.BlockSpec((Z, NBLK1), lambda i: (0, _ia(i))),
        pl.BlockSpec((NBLK1, C), lambda i: (_ia(i), 0)),
        pl.BlockSpec((NBLK1, C), lambda i: (_ib(i), 0)),
        pl.BlockSpec((NBLK1, D), lambda i: (_ia(i), 0)),
        pl.BlockSpec((C, C), lambda i: (0, 1)),
        pl.BlockSpec((C, C), lambda i: (0, 2)),
        pl.BlockSpec((3, C), lambda i: (0, 0)),
        pl.BlockSpec((D, H), lambda i: (0, 0)),
        pl.BlockSpec((C, C), lambda i: (0, 0)),
        pl.BlockSpec((1, C), lambda i: (0, 0)),
    ],
    out_specs=pl.BlockSpec((NBLK1, C), lambda i: (_ib(i), 0)),
    out_shape=jax.ShapeDtypeStruct((N, C), jnp.float32),
    scratch_shapes=[
        pltpu.VMEM((Z, C), jnp.float32),
        pltpu.VMEM((Z, D), jnp.float32),
        pltpu.VMEM((Z, 1), jnp.float32),
        pltpu.VMEM((1, D), jnp.float32),
        pltpu.VMEM((Z, C), jnp.bfloat16),
        pltpu.VMEM((Z, C), jnp.bfloat16),
        pltpu.VMEM((H, Z), jnp.float32),
        pltpu.VMEM((NBLK1, C), jnp.bfloat16),
        pltpu.VMEM((C, C), jnp.bfloat16),
    ],
    compiler_params=pltpu.CompilerParams(
        dimension_semantics=("arbitrary",),
        vmem_limit_bytes=100 * 1024 * 1024),
)


NBLKQ = 1024


def _tcq_body(feat_ref, wq_ref, bqkv_ref, q_ref):
    q_ref[...] = ((lax.dot(feat_ref[...], wq_ref[...],
                           preferred_element_type=jnp.float32)
                   + bqkv_ref[0:1, :]) * SCALE).astype(jnp.bfloat16)


_tcq = pl.pallas_call(
    _tcq_body,
    grid=(N // NBLKQ,),
    in_specs=[
        pl.BlockSpec((NBLKQ, C), lambda i: (i, 0)),
        pl.BlockSpec((C, C), lambda i: (0, 0)),
        pl.BlockSpec((3, C), lambda i: (0, 0)),
    ],
    out_specs=pl.BlockSpec((NBLKQ, C), lambda i: (i, 0)),
    out_shape=jax.ShapeDtypeStruct((N, C), jnp.bfloat16),
    compiler_params=pltpu.CompilerParams(
        dimension_semantics=("arbitrary",)),
)


def kernel(pos, feat, cluster_feat, cluster_score, mean_assignment, mask,
           member_idx, batch_idx, k, valid_row_idx, attend_means,
           W_qkv, b_qkv, W_pos, b_pos, W_proj, b_proj):
    feat2 = feat.reshape(N, C)
    pos2 = pos.reshape(N, D)
    maskf = mask.reshape(N).astype(jnp.float32)
    mi = member_idx.astype(jnp.int32)
    cs = cluster_score.astype(jnp.float32)

    s_flat, msum = _get_sc_build()(mi.reshape(-1), cs.reshape(-1), maskf)

    kk = jnp.asarray(k, jnp.int32).reshape(1, 1)
    bqkv3 = b_qkv.reshape(3, C)
    q2 = _tcq(feat2, W_qkv, bqkv3)
    out = _tc12(kk, msum.reshape(Z, 1), s_flat.reshape(Z, N), feat2, q2,
                pos2, W_qkv, W_qkv, bqkv3, W_pos, W_proj,
                b_proj.reshape(1, C))
    return out.reshape(1, N, C)
